# Initial kernel scaffold; baseline (speedup 1.0000x reference)
import jax
import jax.numpy as jnp
from jax.experimental import pallas as pl


def kernel(x, edge_index, W1, b1, W2, b2):
    raise NotImplementedError("write your pallas kernel here")


# speedup vs baseline: 5.4177x; 5.4177x over previous
import functools

import jax
import jax.numpy as jnp
from jax import lax
from jax.experimental import pallas as pl
from jax.experimental.pallas import tpu as pltpu
from jax.experimental.pallas import tpu_sc as plsc


def _make_agg(N, D, E):
    info = plsc.get_sparse_core_info()
    NC, NS = info.num_cores, info.num_subcores
    NW = NC * NS
    CH = 80
    per_w = E // NW
    n_chunks = per_w // CH
    assert per_w * NW == E and n_chunks * CH == per_w
    ZR = 40
    n_row_chunks = N // ZR
    assert n_row_chunks * ZR == N
    chunks_per_tile = (n_row_chunks + NS - 1) // NS

    mesh = plsc.VectorSubcoreMesh(core_axis_name="c", subcore_axis_name="s")

    @functools.partial(
        pl.kernel,
        out_type=jax.ShapeDtypeStruct((NC, N, D), jnp.float32),
        mesh=mesh,
        scratch_types=[
            pltpu.VMEM((CH,), jnp.int32),
            pltpu.VMEM((CH,), jnp.int32),
            pltpu.VMEM((CH, D), jnp.float32),
            pltpu.VMEM((ZR, D), jnp.float32),
            pltpu.VMEM_SHARED((N, D), jnp.float32),
            pltpu.SemaphoreType.DMA,
        ],
    )
    def agg_kernel(x_hbm, src_hbm, dst_hbm, out_hbm,
                   src_v, dst_v, rows_v, stage_v, acc_sh, sem):
        cid = lax.axis_index("c")
        sid = lax.axis_index("s")
        wid = cid * NS + sid

        zeros16 = jnp.zeros((16,), jnp.float32)

        def zero_body(i, _):
            stage_v[i // (D // 16), pl.ds((i % (D // 16)) * 16, 16)] = zeros16
            return 0

        lax.fori_loop(0, ZR * (D // 16), zero_body, 0)

        for j in range(chunks_per_tile):
            c = sid + j * NS

            @pl.when(c < n_row_chunks)
            def _():
                pltpu.sync_copy(stage_v, acc_sh.at[pl.ds(c * ZR, ZR)])

        plsc.subcore_barrier()

        ebase = wid * per_w

        def body(i, _):
            base = ebase + i * CH
            pltpu.sync_copy(src_hbm.at[pl.ds(base, CH)], src_v)
            pltpu.sync_copy(dst_hbm.at[pl.ds(base, CH)], dst_v)
            pltpu.async_copy(x_hbm.at[src_v], rows_v, sem).wait()
            pltpu.sync_copy(rows_v, acc_sh.at[dst_v], add=True)
            return 0

        lax.fori_loop(0, n_chunks, body, 0)
        plsc.subcore_barrier()

        for j in range(chunks_per_tile):
            c = sid + j * NS

            @pl.when(c < n_row_chunks)
            def _():
                pltpu.sync_copy(acc_sh.at[pl.ds(c * ZR, ZR)], stage_v)
                pltpu.sync_copy(stage_v, out_hbm.at[cid, pl.ds(c * ZR, ZR)])

    return agg_kernel


def _mlp_call(x, p, W1, b1, W2, b2):
    N, D = x.shape
    BLK = 2000
    assert N % BLK == 0

    def mlp_body(x_ref, p0_ref, p1_ref, w1_ref, b1_ref, w2_ref, b2_ref,
                 o_ref):
        h = x_ref[...] + p0_ref[...] + p1_ref[...]
        h = jnp.dot(h, w1_ref[...], preferred_element_type=jnp.float32)
        h = jnp.maximum(h + b1_ref[...], 0.0)
        h = jnp.dot(h, w2_ref[...], preferred_element_type=jnp.float32)
        o_ref[...] = h + b2_ref[...]

    return pl.pallas_call(
        mlp_body,
        grid=(N // BLK,),
        in_specs=[
            pl.BlockSpec((BLK, D), lambda i: (i, 0)),
            pl.BlockSpec((BLK, D), lambda i: (i, 0)),
            pl.BlockSpec((BLK, D), lambda i: (i, 0)),
            pl.BlockSpec((D, D), lambda i: (0, 0)),
            pl.BlockSpec((1, D), lambda i: (0, 0)),
            pl.BlockSpec((D, D), lambda i: (0, 0)),
            pl.BlockSpec((1, D), lambda i: (0, 0)),
        ],
        out_specs=pl.BlockSpec((BLK, D), lambda i: (i, 0)),
        out_shape=jax.ShapeDtypeStruct((N, D), jnp.float32),
    )(x, p[0], p[1], W1, b1.reshape(1, D), W2, b2.reshape(1, D))


def kernel(x, edge_index, W1, b1, W2, b2):
    N, D = x.shape
    E = edge_index.shape[1]
    src = edge_index[0].astype(jnp.int32)
    dst = edge_index[1].astype(jnp.int32)
    p = _make_agg(N, D, E)(x, src, dst)
    return _mlp_call(x, p, W1, b1, W2, b2)
